# baseline (speedup 1.0000x reference)
import functools

import jax
import jax.numpy as jnp
from jax.experimental import pallas as pl
from jax.experimental.pallas import tpu as pltpu


def _ft_kernel(x_ref, w_ref, o_ref):
    o_ref[...] = jnp.dot(x_ref[...], w_ref[...],
                         preferred_element_type=jnp.float32)


def _pass1_kernel(adj_ref, s_ref, b1_ref, w2_ref, o_ref):
    h = jnp.dot(adj_ref[...], s_ref[...], preferred_element_type=jnp.float32)
    h = jnp.maximum(h + b1_ref[...], 0.0)
    o_ref[...] = jnp.dot(h, w2_ref[...], preferred_element_type=jnp.float32)


def _pass2_kernel(adj_ref, t_ref, b2_ref, wl1_ref, bl1_ref, wl2_ref, bl2_ref,
                  wl3_ref, bl3_ref, o_ref):
    h = jnp.dot(adj_ref[...], t_ref[...], preferred_element_type=jnp.float32)
    h = jnp.maximum(h + b2_ref[...], 0.0)
    h = jnp.maximum(jnp.dot(h, wl1_ref[...],
                            preferred_element_type=jnp.float32) + bl1_ref[...],
                    0.0)
    h = jnp.maximum(jnp.dot(h, wl2_ref[...],
                            preferred_element_type=jnp.float32) + bl2_ref[...],
                    0.0)
    logits = jnp.dot(h, wl3_ref[...],
                     preferred_element_type=jnp.float32) + bl3_ref[...]
    m = jnp.max(logits, axis=-1, keepdims=True)
    lse = m + jnp.log(jnp.sum(jnp.exp(logits - m), axis=-1, keepdims=True))
    o_ref[...] = logits - lse


def _pick_bm(n):
    for bm in (400, 200, 80, 40, 8):
        if n % bm == 0:
            return bm
    return n


@functools.partial(jax.jit, static_argnames=())
def kernel(x, adj, W_gc1, b_gc1, W_gc2, b_gc2, W_l1, b_l1, W_l2, b_l2,
           W_l3, b_l3):
    _, n, in_f = x.shape
    hid = W_gc1.shape[1]
    hid2 = W_l2.shape[1]
    classes = W_l3.shape[1]
    x2 = x.reshape(n, in_f)
    adj2 = adj.reshape(n, n)
    bm = _pick_bm(n)
    grid = (n // bm,)

    s1 = pl.pallas_call(
        _ft_kernel,
        grid=grid,
        in_specs=[
            pl.BlockSpec((bm, in_f), lambda i: (i, 0)),
            pl.BlockSpec((in_f, hid), lambda i: (0, 0)),
        ],
        out_specs=pl.BlockSpec((bm, hid), lambda i: (i, 0)),
        out_shape=jax.ShapeDtypeStruct((n, hid), jnp.float32),
    )(x2, W_gc1)

    t = pl.pallas_call(
        _pass1_kernel,
        grid=grid,
        in_specs=[
            pl.BlockSpec((bm, n), lambda i: (i, 0)),
            pl.BlockSpec((n, hid), lambda i: (0, 0)),
            pl.BlockSpec((1, hid), lambda i: (0, 0)),
            pl.BlockSpec((hid, hid), lambda i: (0, 0)),
        ],
        out_specs=pl.BlockSpec((bm, hid), lambda i: (i, 0)),
        out_shape=jax.ShapeDtypeStruct((n, hid), jnp.float32),
        compiler_params=pltpu.CompilerParams(
            dimension_semantics=("arbitrary",)),
    )(adj2, s1, b_gc1.reshape(1, hid), W_gc2)

    out_nc = pl.pallas_call(
        _pass2_kernel,
        grid=grid,
        in_specs=[
            pl.BlockSpec((bm, n), lambda i: (i, 0)),
            pl.BlockSpec((n, hid), lambda i: (0, 0)),
            pl.BlockSpec((1, hid), lambda i: (0, 0)),
            pl.BlockSpec((hid, hid), lambda i: (0, 0)),
            pl.BlockSpec((1, hid), lambda i: (0, 0)),
            pl.BlockSpec((hid, hid2), lambda i: (0, 0)),
            pl.BlockSpec((1, hid2), lambda i: (0, 0)),
            pl.BlockSpec((hid2, classes), lambda i: (0, 0)),
            pl.BlockSpec((1, classes), lambda i: (0, 0)),
        ],
        out_specs=pl.BlockSpec((bm, classes), lambda i: (i, 0)),
        out_shape=jax.ShapeDtypeStruct((n, classes), jnp.float32),
        compiler_params=pltpu.CompilerParams(
            dimension_semantics=("arbitrary",)),
    )(adj2, t, b_gc2.reshape(1, hid), W_l1, b_l1.reshape(1, hid),
      W_l2, b_l2.reshape(1, hid2), W_l3, b_l3.reshape(1, classes))

    return jnp.transpose(out_nc)[None]
